# Initial kernel scaffold; baseline (speedup 1.0000x reference)
import jax
import jax.numpy as jnp
from jax.experimental import pallas as pl


def kernel(x, edge_index, batch, z, Wl1, b1, Wr1, Wl2, b2, Wr2, Wfc1, bfc1, Wfc2, bfc2, Wp, bp):
    raise NotImplementedError("write your pallas kernel here")


# speedup vs baseline: 4.3190x; 4.3190x over previous
import functools

import jax
import jax.numpy as jnp
from jax import lax
from jax.experimental import pallas as pl
from jax.experimental.pallas import tpu as pltpu
from jax.experimental.pallas import tpu_sc as plsc

N = 10000
E = 320000
D = 128
H = 128
FC = 128
B = 64
SF = 6

NC = 2
NS = 16
NW = NC * NS
EPW = E // NW
K = 80
NCHUNK = EPW // K
RPT = 624
RTAIL = N - NS * RPT
CW = 16
DC = D + CW
SCH = 48
NSCH = RPT // SCH

_mesh = plsc.VectorSubcoreMesh(core_axis_name="c", subcore_axis_name="s")


def _make_sc_agg(width):
    @functools.partial(
        pl.kernel,
        out_type=jax.ShapeDtypeStruct((NC * N, width), jnp.float32),
        mesh=_mesh,
        scratch_types=[
            pltpu.VMEM((K,), jnp.int32),
            pltpu.VMEM((K,), jnp.int32),
            pltpu.VMEM((K, width), jnp.float32),
            pltpu.VMEM((SCH, width), jnp.float32),
            pltpu.VMEM_SHARED((N, width), jnp.float32),
            pltpu.SemaphoreType.DMA,
        ],
    )
    def _sc_agg(table, src, dst, zinit,
                agg_out,
                sidx, didx, rows, stage, aggs, sem):
        cid = lax.axis_index("c")
        sid = lax.axis_index("s")
        wid = cid * NS + sid
        rbase = sid * RPT
        pltpu.sync_copy(zinit.at[pl.ds(0, SCH)], stage)
        for j in range(NSCH):
            pltpu.sync_copy(stage, aggs.at[pl.ds(rbase + j * SCH, SCH)])

        @pl.when(sid == NS - 1)
        def _tail_init():
            t0 = NS * RPT
            pltpu.sync_copy(stage.at[pl.ds(0, RTAIL)],
                            aggs.at[pl.ds(t0, RTAIL)])

        plsc.subcore_barrier()
        e0 = wid * EPW

        def step(i, carry):
            b = e0 + i * K
            pltpu.sync_copy(src.at[pl.ds(b, K)], sidx)
            pltpu.sync_copy(dst.at[pl.ds(b, K)], didx)
            pltpu.async_copy(table.at[sidx], rows, sem).wait()
            pltpu.sync_copy(rows, aggs.at[didx], add=True)
            return carry

        lax.fori_loop(0, NCHUNK, step, 0)
        plsc.subcore_barrier()
        obase = cid * N
        for j in range(NSCH):
            o = rbase + j * SCH
            pltpu.sync_copy(aggs.at[pl.ds(o, SCH)], stage)
            pltpu.sync_copy(stage, agg_out.at[pl.ds(obase + o, SCH)])

        @pl.when(sid == NS - 1)
        def _tail_out():
            t0 = NS * RPT
            pltpu.sync_copy(aggs.at[pl.ds(t0, RTAIL)],
                            stage.at[pl.ds(0, RTAIL)])
            pltpu.sync_copy(stage.at[pl.ds(0, RTAIL)],
                            agg_out.at[pl.ds(obase + t0, RTAIL)])

    return _sc_agg


_sc_agg = _make_sc_agg(D)


@functools.partial(
    pl.kernel,
    out_type=jax.ShapeDtypeStruct((NC * N, D), jnp.float32),
    mesh=_mesh,
    scratch_types=[
        pltpu.VMEM((K,), jnp.int32),
        pltpu.VMEM((K, D), jnp.float32),
        pltpu.VMEM((SCH, D), jnp.float32),
        pltpu.VMEM_SHARED((N, D), jnp.float32),
    ],
)
def _sc_cnt(dst, z128, ones_h,
            cnt_out,
            didx, ones_v, cstage, cnts):
    cid = lax.axis_index("c")
    sid = lax.axis_index("s")
    wid = cid * NS + sid
    rbase = sid * RPT
    pltpu.sync_copy(z128.at[pl.ds(0, SCH)], cstage)
    for j in range(NSCH):
        pltpu.sync_copy(cstage, cnts.at[pl.ds(rbase + j * SCH, SCH)])

    @pl.when(sid == NS - 1)
    def _tail_init():
        t0 = NS * RPT
        pltpu.sync_copy(cstage.at[pl.ds(0, RTAIL)], cnts.at[pl.ds(t0, RTAIL)])

    pltpu.sync_copy(ones_h, ones_v)
    plsc.subcore_barrier()
    e0 = wid * EPW

    def step(i, carry):
        b = e0 + i * K
        pltpu.sync_copy(dst.at[pl.ds(b, K)], didx)
        pltpu.sync_copy(ones_v, cnts.at[didx], add=True)
        return carry

    lax.fori_loop(0, NCHUNK, step, 0)
    plsc.subcore_barrier()
    obase = cid * N
    for j in range(NSCH):
        o = rbase + j * SCH
        pltpu.sync_copy(cnts.at[pl.ds(o, SCH)], cstage)
        pltpu.sync_copy(cstage, cnt_out.at[pl.ds(obase + o, SCH)])

    @pl.when(sid == NS - 1)
    def _tail_out():
        t0 = NS * RPT
        pltpu.sync_copy(cnts.at[pl.ds(t0, RTAIL)], cstage.at[pl.ds(0, RTAIL)])
        pltpu.sync_copy(cstage.at[pl.ds(0, RTAIL)],
                        cnt_out.at[pl.ds(obase + t0, RTAIL)])


R = 400
G = N // R


def _dotT(a, w):
    return lax.dot_general(a, w, (((1,), (1,)), ((), ())),
                           preferred_element_type=jnp.float32)


def _dense1_body(xr, aggr, cntr, wl, wr, br, hr):
    agg = aggr[0] + aggr[1]
    c = cntr[0][:, 0:1] + cntr[1][:, 0:1]
    mean = agg / jnp.maximum(c, 1.0)
    o = _dotT(mean, wl[...]) + _dotT(xr[...], wr[...]) + br[...]
    n2 = jnp.sum(o * o, axis=1, keepdims=True)
    o = o * lax.rsqrt(jnp.maximum(n2, 1e-24))
    hr[...] = jnp.maximum(o, 0.0)


def _tc_dense1(x, aggp, cntp, Wl, Wr, b2d):
    return pl.pallas_call(
        _dense1_body,
        grid=(G,),
        in_specs=[
            pl.BlockSpec((R, D), lambda i: (i, 0)),
            pl.BlockSpec((NC, R, D), lambda i: (0, i, 0)),
            pl.BlockSpec((NC, R, D), lambda i: (0, i, 0)),
            pl.BlockSpec((H, D), lambda i: (0, 0)),
            pl.BlockSpec((H, D), lambda i: (0, 0)),
            pl.BlockSpec((1, H), lambda i: (0, 0)),
        ],
        out_specs=pl.BlockSpec((R, H), lambda i: (i, 0)),
        out_shape=jax.ShapeDtypeStruct((N, H), jnp.float32),
    )(x, aggp, cntp, Wl, Wr, b2d)


def _dense2_body(hr, aggr, cntr, batr, zr, wl, wr, br,
                 w1g, w1z, b1r, w2, b2r, wp, bpr, outr, pooled):
    i = pl.program_id(0)

    @pl.when(i == 0)
    def _init():
        pooled[...] = jnp.zeros_like(pooled)

    agg = aggr[0] + aggr[1]
    c = cntr[0][:, 0:1] + cntr[1][:, 0:1]
    mean = agg / jnp.maximum(c, 1.0)
    o = _dotT(mean, wl[...]) + _dotT(hr[...], wr[...]) + br[...]
    n2 = jnp.sum(o * o, axis=1, keepdims=True)
    o = o * lax.rsqrt(jnp.maximum(n2, 1e-24))
    h2 = jnp.maximum(o, 0.0)

    oh = (batr[...] == lax.broadcasted_iota(jnp.int32, (R, B), 1)
          ).astype(jnp.float32)
    pooled[...] += lax.dot_general(oh, h2, (((0,), (0,)), ((), ())),
                                   preferred_element_type=jnp.float32)

    @pl.when(i == G - 1)
    def _head():
        g = pooled[...]
        f = jnp.maximum(_dotT(g, w1g[...]) + _dotT(zr[...], w1z[...])
                        + b1r[...], 0.0)
        f = jnp.maximum(_dotT(f, w2[...]) + b2r[...], 0.0)
        out = jnp.sum(f * wp[...], axis=1, keepdims=True) + bpr[...]
        t = -out
        outr[...] = jnp.maximum(t, 0.0) + jnp.log(1.0 + jnp.exp(-jnp.abs(t)))


def _tc_dense2(h1, aggp, cntp, bat2d, z, Wl, Wr, b2d,
               W1g, W1z, bfc1_2d, Wfc2, bfc2_2d, Wp, bp2d):
    return pl.pallas_call(
        _dense2_body,
        grid=(G,),
        in_specs=[
            pl.BlockSpec((R, H), lambda i: (i, 0)),
            pl.BlockSpec((NC, R, H), lambda i: (0, i, 0)),
            pl.BlockSpec((NC, R, D), lambda i: (0, i, 0)),
            pl.BlockSpec((R, 1), lambda i: (i, 0)),
            pl.BlockSpec((B, SF), lambda i: (0, 0)),
            pl.BlockSpec((H, H), lambda i: (0, 0)),
            pl.BlockSpec((H, H), lambda i: (0, 0)),
            pl.BlockSpec((1, H), lambda i: (0, 0)),
            pl.BlockSpec((FC, H), lambda i: (0, 0)),
            pl.BlockSpec((FC, SF), lambda i: (0, 0)),
            pl.BlockSpec((1, FC), lambda i: (0, 0)),
            pl.BlockSpec((FC, FC), lambda i: (0, 0)),
            pl.BlockSpec((1, FC), lambda i: (0, 0)),
            pl.BlockSpec((1, FC), lambda i: (0, 0)),
            pl.BlockSpec((B, 1), lambda i: (0, 0)),
        ],
        out_specs=pl.BlockSpec((B, 1), lambda i: (0, 0)),
        out_shape=jax.ShapeDtypeStruct((B, 1), jnp.float32),
        scratch_shapes=[pltpu.VMEM((B, H), jnp.float32)],
    )(h1, aggp, cntp, bat2d, z, Wl, Wr, b2d,
      W1g, W1z, bfc1_2d, Wfc2, bfc2_2d, Wp, bp2d)


def kernel(x, edge_index, batch, z, Wl1, b1, Wr1, Wl2, b2, Wr2,
           Wfc1, bfc1, Wfc2, bfc2, Wp, bp):
    src = edge_index[0]
    dst = edge_index[1]
    x = x.astype(jnp.float32)
    z128 = jnp.zeros((N, D), jnp.float32)
    ones_h = jnp.ones((K, D), jnp.float32)

    cntp = _sc_cnt(dst, z128, ones_h).reshape(NC, N, D)
    aggp1 = _sc_agg(x, src, dst, z128).reshape(NC, N, D)
    h1 = _tc_dense1(x, aggp1, cntp, Wl1, Wr1, b1.reshape(1, H))
    aggp2 = _sc_agg(h1, src, dst, z128).reshape(NC, N, D)
    out = _tc_dense2(
        h1, aggp2, cntp, batch.reshape(N, 1), z, Wl2, Wr2, b2.reshape(1, H),
        Wfc1[:, :H], Wfc1[:, H:], bfc1.reshape(1, FC),
        Wfc2, bfc2.reshape(1, FC), Wp,
        jnp.broadcast_to(bp.reshape(1, 1), (B, 1)))
    return out


# speedup vs baseline: 5.7461x; 1.3304x over previous
import functools

import jax
import jax.numpy as jnp
from jax import lax
from jax.experimental import pallas as pl
from jax.experimental.pallas import tpu as pltpu
from jax.experimental.pallas import tpu_sc as plsc

N = 10000
E = 320000
D = 128
H = 128
FC = 128
B = 64
SF = 6

NC = 2
NS = 16
NW = NC * NS
EPW = E // NW
K = 40
NCHUNK = EPW // K
RPT = 624
RTAIL = N - NS * RPT
SCH = 24
NSCH = RPT // SCH

_mesh = plsc.VectorSubcoreMesh(core_axis_name="c", subcore_axis_name="s")


def _make_sc_agg(width):
    @functools.partial(
        pl.kernel,
        out_type=jax.ShapeDtypeStruct((NC * N, width), jnp.float32),
        mesh=_mesh,
        scratch_types=[
            pltpu.VMEM((K,), jnp.int32),
            pltpu.VMEM((K,), jnp.int32),
            pltpu.VMEM((K,), jnp.int32),
            pltpu.VMEM((K,), jnp.int32),
            pltpu.VMEM((K, width), jnp.float32),
            pltpu.VMEM((K, width), jnp.float32),
            pltpu.VMEM_SHARED((N, width), jnp.float32),
            pltpu.SemaphoreType.DMA,
            pltpu.SemaphoreType.DMA,
            pltpu.SemaphoreType.DMA,
            pltpu.SemaphoreType.DMA,
        ],
    )
    def _sc_agg(table, src, dst, zinit,
                agg_out,
                sidxa, didxa, sidxb, didxb, rowsa, rowsb, aggs,
                semia, semib, semra, semrb):
        cid = lax.axis_index("c")
        sid = lax.axis_index("s")
        wid = cid * NS + sid
        rbase = sid * RPT
        stage = rowsa.at[pl.ds(0, SCH)]
        pltpu.sync_copy(zinit.at[pl.ds(0, SCH)], stage)
        for j in range(NSCH):
            pltpu.sync_copy(stage, aggs.at[pl.ds(rbase + j * SCH, SCH)])

        @pl.when(sid == NS - 1)
        def _tail_init():
            t0 = NS * RPT
            pltpu.sync_copy(rowsa.at[pl.ds(0, RTAIL)],
                            aggs.at[pl.ds(t0, RTAIL)])

        plsc.subcore_barrier()
        e0 = wid * EPW
        dmyr = table.at[pl.ds(0, K)]
        dmyi = src.at[pl.ds(0, K)]

        def loadidx(c, si, di, sem):
            b = e0 + c * K
            pltpu.async_copy(src.at[pl.ds(b, K)], si, sem)
            pltpu.async_copy(dst.at[pl.ds(b, K)], di, sem)

        def waitidx(si, di, sem):
            pltpu.make_async_copy(dmyi, si, sem).wait()
            pltpu.make_async_copy(dmyi, di, sem).wait()

        loadidx(0, sidxa, didxa, semia)
        loadidx(1, sidxb, didxb, semib)
        waitidx(sidxa, didxa, semia)
        pltpu.async_copy(table.at[sidxa], rowsa, semra)

        def pair(j, carry):
            c = 2 * j
            waitidx(sidxb, didxb, semib)
            pltpu.async_copy(table.at[sidxb], rowsb, semrb)
            pltpu.make_async_copy(dmyr, rowsa, semra).wait()
            pltpu.sync_copy(rowsa, aggs.at[didxa], add=True)
            loadidx(c + 2, sidxa, didxa, semia)
            waitidx(sidxa, didxa, semia)
            pltpu.async_copy(table.at[sidxa], rowsa, semra)
            pltpu.make_async_copy(dmyr, rowsb, semrb).wait()
            pltpu.sync_copy(rowsb, aggs.at[didxb], add=True)
            loadidx(c + 3, sidxb, didxb, semib)
            return carry

        lax.fori_loop(0, (NCHUNK - 2) // 2, pair, 0)
        waitidx(sidxb, didxb, semib)
        pltpu.async_copy(table.at[sidxb], rowsb, semrb)
        pltpu.make_async_copy(dmyr, rowsa, semra).wait()
        pltpu.sync_copy(rowsa, aggs.at[didxa], add=True)
        pltpu.make_async_copy(dmyr, rowsb, semrb).wait()
        pltpu.sync_copy(rowsb, aggs.at[didxb], add=True)
        plsc.subcore_barrier()
        obase = cid * N
        for j in range(NSCH):
            o = rbase + j * SCH
            pltpu.sync_copy(aggs.at[pl.ds(o, SCH)], stage)
            pltpu.sync_copy(stage, agg_out.at[pl.ds(obase + o, SCH)])

        @pl.when(sid == NS - 1)
        def _tail_out():
            t0 = NS * RPT
            pltpu.sync_copy(aggs.at[pl.ds(t0, RTAIL)],
                            rowsa.at[pl.ds(0, RTAIL)])
            pltpu.sync_copy(rowsa.at[pl.ds(0, RTAIL)],
                            agg_out.at[pl.ds(obase + t0, RTAIL)])

    return _sc_agg


_sc_agg = _make_sc_agg(D)


@functools.partial(
    pl.kernel,
    out_type=jax.ShapeDtypeStruct((NC * N, D), jnp.float32),
    mesh=_mesh,
    scratch_types=[
        pltpu.VMEM((K,), jnp.int32),
        pltpu.VMEM((K,), jnp.int32),
        pltpu.VMEM((K, D), jnp.float32),
        pltpu.VMEM((SCH, D), jnp.float32),
        pltpu.VMEM_SHARED((N, D), jnp.float32),
        pltpu.SemaphoreType.DMA,
        pltpu.SemaphoreType.DMA,
    ],
)
def _sc_cnt(dst, z128, ones_h,
            cnt_out,
            didxa, didxb, ones_v, cstage, cnts, semia, semib):
    cid = lax.axis_index("c")
    sid = lax.axis_index("s")
    wid = cid * NS + sid
    rbase = sid * RPT
    pltpu.sync_copy(z128.at[pl.ds(0, SCH)], cstage)
    for j in range(NSCH):
        pltpu.sync_copy(cstage, cnts.at[pl.ds(rbase + j * SCH, SCH)])

    @pl.when(sid == NS - 1)
    def _tail_init():
        t0 = NS * RPT
        pltpu.sync_copy(cstage.at[pl.ds(0, RTAIL)], cnts.at[pl.ds(t0, RTAIL)])

    pltpu.sync_copy(ones_h, ones_v)
    plsc.subcore_barrier()
    e0 = wid * EPW
    dmyi = dst.at[pl.ds(0, K)]

    pltpu.async_copy(dst.at[pl.ds(e0, K)], didxa, semia)
    pltpu.async_copy(dst.at[pl.ds(e0 + K, K)], didxb, semib)

    def pair(j, carry):
        c = 2 * j
        pltpu.make_async_copy(dmyi, didxa, semia).wait()
        pltpu.sync_copy(ones_v, cnts.at[didxa], add=True)
        pltpu.async_copy(dst.at[pl.ds(e0 + (c + 2) * K, K)], didxa, semia)
        pltpu.make_async_copy(dmyi, didxb, semib).wait()
        pltpu.sync_copy(ones_v, cnts.at[didxb], add=True)
        pltpu.async_copy(dst.at[pl.ds(e0 + (c + 3) * K, K)], didxb, semib)
        return carry

    lax.fori_loop(0, (NCHUNK - 2) // 2, pair, 0)
    pltpu.make_async_copy(dmyi, didxa, semia).wait()
    pltpu.sync_copy(ones_v, cnts.at[didxa], add=True)
    pltpu.make_async_copy(dmyi, didxb, semib).wait()
    pltpu.sync_copy(ones_v, cnts.at[didxb], add=True)
    plsc.subcore_barrier()
    obase = cid * N
    for j in range(NSCH):
        o = rbase + j * SCH
        pltpu.sync_copy(cnts.at[pl.ds(o, SCH)], cstage)
        pltpu.sync_copy(cstage, cnt_out.at[pl.ds(obase + o, SCH)])

    @pl.when(sid == NS - 1)
    def _tail_out():
        t0 = NS * RPT
        pltpu.sync_copy(cnts.at[pl.ds(t0, RTAIL)], cstage.at[pl.ds(0, RTAIL)])
        pltpu.sync_copy(cstage.at[pl.ds(0, RTAIL)],
                        cnt_out.at[pl.ds(obase + t0, RTAIL)])


R = 400
G = N // R


def _dotT(a, w):
    return lax.dot_general(a, w, (((1,), (1,)), ((), ())),
                           preferred_element_type=jnp.float32)


def _dense1_body(xr, aggr, cntr, wl, wr, br, hr):
    agg = aggr[0] + aggr[1]
    c = cntr[0][:, 0:1] + cntr[1][:, 0:1]
    mean = agg / jnp.maximum(c, 1.0)
    o = _dotT(mean, wl[...]) + _dotT(xr[...], wr[...]) + br[...]
    n2 = jnp.sum(o * o, axis=1, keepdims=True)
    o = o * lax.rsqrt(jnp.maximum(n2, 1e-24))
    hr[...] = jnp.maximum(o, 0.0)


def _tc_dense1(x, aggp, cntp, Wl, Wr, b2d):
    return pl.pallas_call(
        _dense1_body,
        grid=(G,),
        in_specs=[
            pl.BlockSpec((R, D), lambda i: (i, 0)),
            pl.BlockSpec((NC, R, D), lambda i: (0, i, 0)),
            pl.BlockSpec((NC, R, D), lambda i: (0, i, 0)),
            pl.BlockSpec((H, D), lambda i: (0, 0)),
            pl.BlockSpec((H, D), lambda i: (0, 0)),
            pl.BlockSpec((1, H), lambda i: (0, 0)),
        ],
        out_specs=pl.BlockSpec((R, H), lambda i: (i, 0)),
        out_shape=jax.ShapeDtypeStruct((N, H), jnp.float32),
    )(x, aggp, cntp, Wl, Wr, b2d)


def _dense2_body(hr, aggr, cntr, batr, zr, wl, wr, br,
                 w1g, w1z, b1r, w2, b2r, wp, bpr, outr, pooled):
    i = pl.program_id(0)

    @pl.when(i == 0)
    def _init():
        pooled[...] = jnp.zeros_like(pooled)

    agg = aggr[0] + aggr[1]
    c = cntr[0][:, 0:1] + cntr[1][:, 0:1]
    mean = agg / jnp.maximum(c, 1.0)
    o = _dotT(mean, wl[...]) + _dotT(hr[...], wr[...]) + br[...]
    n2 = jnp.sum(o * o, axis=1, keepdims=True)
    o = o * lax.rsqrt(jnp.maximum(n2, 1e-24))
    h2 = jnp.maximum(o, 0.0)

    oh = (batr[...] == lax.broadcasted_iota(jnp.int32, (R, B), 1)
          ).astype(jnp.float32)
    pooled[...] += lax.dot_general(oh, h2, (((0,), (0,)), ((), ())),
                                   preferred_element_type=jnp.float32)

    @pl.when(i == G - 1)
    def _head():
        g = pooled[...]
        f = jnp.maximum(_dotT(g, w1g[...]) + _dotT(zr[...], w1z[...])
                        + b1r[...], 0.0)
        f = jnp.maximum(_dotT(f, w2[...]) + b2r[...], 0.0)
        out = jnp.sum(f * wp[...], axis=1, keepdims=True) + bpr[...]
        t = -out
        outr[...] = jnp.maximum(t, 0.0) + jnp.log(1.0 + jnp.exp(-jnp.abs(t)))


def _tc_dense2(h1, aggp, cntp, bat2d, z, Wl, Wr, b2d,
               W1g, W1z, bfc1_2d, Wfc2, bfc2_2d, Wp, bp2d):
    return pl.pallas_call(
        _dense2_body,
        grid=(G,),
        in_specs=[
            pl.BlockSpec((R, H), lambda i: (i, 0)),
            pl.BlockSpec((NC, R, H), lambda i: (0, i, 0)),
            pl.BlockSpec((NC, R, D), lambda i: (0, i, 0)),
            pl.BlockSpec((R, 1), lambda i: (i, 0)),
            pl.BlockSpec((B, SF), lambda i: (0, 0)),
            pl.BlockSpec((H, H), lambda i: (0, 0)),
            pl.BlockSpec((H, H), lambda i: (0, 0)),
            pl.BlockSpec((1, H), lambda i: (0, 0)),
            pl.BlockSpec((FC, H), lambda i: (0, 0)),
            pl.BlockSpec((FC, SF), lambda i: (0, 0)),
            pl.BlockSpec((1, FC), lambda i: (0, 0)),
            pl.BlockSpec((FC, FC), lambda i: (0, 0)),
            pl.BlockSpec((1, FC), lambda i: (0, 0)),
            pl.BlockSpec((1, FC), lambda i: (0, 0)),
            pl.BlockSpec((B, 1), lambda i: (0, 0)),
        ],
        out_specs=pl.BlockSpec((B, 1), lambda i: (0, 0)),
        out_shape=jax.ShapeDtypeStruct((B, 1), jnp.float32),
        scratch_shapes=[pltpu.VMEM((B, H), jnp.float32)],
    )(h1, aggp, cntp, bat2d, z, Wl, Wr, b2d,
      W1g, W1z, bfc1_2d, Wfc2, bfc2_2d, Wp, bp2d)


def kernel(x, edge_index, batch, z, Wl1, b1, Wr1, Wl2, b2, Wr2,
           Wfc1, bfc1, Wfc2, bfc2, Wp, bp):
    src = edge_index[0]
    dst = edge_index[1]
    x = x.astype(jnp.float32)
    z128 = jnp.zeros((N, D), jnp.float32)
    ones_h = jnp.ones((K, D), jnp.float32)

    cntp = _sc_cnt(dst, z128, ones_h).reshape(NC, N, D)
    aggp1 = _sc_agg(x, src, dst, z128).reshape(NC, N, D)
    h1 = _tc_dense1(x, aggp1, cntp, Wl1, Wr1, b1.reshape(1, H))
    aggp2 = _sc_agg(h1, src, dst, z128).reshape(NC, N, D)
    out = _tc_dense2(
        h1, aggp2, cntp, batch.reshape(N, 1), z, Wl2, Wr2, b2.reshape(1, H),
        Wfc1[:, :H], Wfc1[:, H:], bfc1.reshape(1, FC),
        Wfc2, bfc2.reshape(1, FC), Wp,
        jnp.broadcast_to(bp.reshape(1, 1), (B, 1)))
    return out


# speedup vs baseline: 6.9739x; 1.2137x over previous
import functools

import jax
import jax.numpy as jnp
from jax import lax
from jax.experimental import pallas as pl
from jax.experimental.pallas import tpu as pltpu
from jax.experimental.pallas import tpu_sc as plsc

N = 10000
E = 320000
D = 128
H = 128
FC = 128
B = 64
SF = 6

NC = 2
NS = 16
NW = NC * NS
EPW = E // NW
K = 40
NCHUNK = EPW // K
RPT = 624
RTAIL = N - NS * RPT
SCH = 24
NSCH = RPT // SCH

_mesh = plsc.VectorSubcoreMesh(core_axis_name="c", subcore_axis_name="s")


def _make_sc_agg(width):
    @functools.partial(
        pl.kernel,
        out_type=jax.ShapeDtypeStruct((NC * N, width), jnp.float32),
        mesh=_mesh,
        scratch_types=[
            pltpu.VMEM((EPW,), jnp.int32),
            pltpu.VMEM((K,), jnp.int32),
            pltpu.VMEM((K,), jnp.int32),
            pltpu.VMEM((K, width), jnp.float32),
            pltpu.VMEM((K, width), jnp.float32),
            pltpu.VMEM_SHARED((N, width), jnp.float32),
            pltpu.SemaphoreType.DMA,
            pltpu.SemaphoreType.DMA,
            pltpu.SemaphoreType.DMA,
            pltpu.SemaphoreType.DMA,
        ],
    )
    def _sc_agg(table, src, dst, zinit,
                agg_out,
                sidxall, didxa, didxb, rowsa, rowsb, aggs,
                semra, semrb, semia, semib):
        cid = lax.axis_index("c")
        sid = lax.axis_index("s")
        wid = cid * NS + sid
        rbase = sid * RPT
        stage = rowsa.at[pl.ds(0, SCH)]
        pltpu.sync_copy(zinit.at[pl.ds(0, SCH)], stage)
        for j in range(NSCH):
            pltpu.sync_copy(stage, aggs.at[pl.ds(rbase + j * SCH, SCH)])

        @pl.when(sid == NS - 1)
        def _tail_init():
            t0 = NS * RPT
            pltpu.sync_copy(rowsa.at[pl.ds(0, RTAIL)],
                            aggs.at[pl.ds(t0, RTAIL)])

        e0 = wid * EPW
        pltpu.sync_copy(src.at[pl.ds(e0, EPW)], sidxall)
        plsc.subcore_barrier()
        dmyr = table.at[pl.ds(0, K)]
        dmyi = dst.at[pl.ds(0, K)]

        pltpu.async_copy(dst.at[pl.ds(e0, K)], didxa, semia)
        pltpu.async_copy(dst.at[pl.ds(e0 + K, K)], didxb, semib)
        pltpu.async_copy(table.at[sidxall.at[pl.ds(0, K)]], rowsa, semra)

        def pair(j, carry):
            c = 2 * j
            pltpu.async_copy(table.at[sidxall.at[pl.ds((c + 1) * K, K)]],
                             rowsb, semrb)
            pltpu.make_async_copy(dmyr, rowsa, semra).wait()
            pltpu.make_async_copy(dmyi, didxa, semia).wait()
            pltpu.sync_copy(rowsa, aggs.at[didxa], add=True)
            pltpu.async_copy(table.at[sidxall.at[pl.ds((c + 2) * K, K)]],
                             rowsa, semra)
            pltpu.async_copy(dst.at[pl.ds(e0 + (c + 2) * K, K)], didxa, semia)
            pltpu.make_async_copy(dmyr, rowsb, semrb).wait()
            pltpu.make_async_copy(dmyi, didxb, semib).wait()
            pltpu.sync_copy(rowsb, aggs.at[didxb], add=True)
            pltpu.async_copy(dst.at[pl.ds(e0 + (c + 3) * K, K)], didxb, semib)
            return carry

        lax.fori_loop(0, (NCHUNK - 2) // 2, pair, 0)
        pltpu.async_copy(table.at[sidxall.at[pl.ds((NCHUNK - 1) * K, K)]],
                         rowsb, semrb)
        pltpu.make_async_copy(dmyr, rowsa, semra).wait()
        pltpu.make_async_copy(dmyi, didxa, semia).wait()
        pltpu.sync_copy(rowsa, aggs.at[didxa], add=True)
        pltpu.make_async_copy(dmyr, rowsb, semrb).wait()
        pltpu.make_async_copy(dmyi, didxb, semib).wait()
        pltpu.sync_copy(rowsb, aggs.at[didxb], add=True)
        plsc.subcore_barrier()
        obase = cid * N
        for j in range(NSCH):
            o = rbase + j * SCH
            pltpu.sync_copy(aggs.at[pl.ds(o, SCH)], stage)
            pltpu.sync_copy(stage, agg_out.at[pl.ds(obase + o, SCH)])

        @pl.when(sid == NS - 1)
        def _tail_out():
            t0 = NS * RPT
            pltpu.sync_copy(aggs.at[pl.ds(t0, RTAIL)],
                            rowsa.at[pl.ds(0, RTAIL)])
            pltpu.sync_copy(rowsa.at[pl.ds(0, RTAIL)],
                            agg_out.at[pl.ds(obase + t0, RTAIL)])

    return _sc_agg


_sc_agg = _make_sc_agg(D)


@functools.partial(
    pl.kernel,
    out_type=jax.ShapeDtypeStruct((NC * N, D), jnp.float32),
    mesh=_mesh,
    scratch_types=[
        pltpu.VMEM((K,), jnp.int32),
        pltpu.VMEM((K,), jnp.int32),
        pltpu.VMEM((K, D), jnp.float32),
        pltpu.VMEM((SCH, D), jnp.float32),
        pltpu.VMEM_SHARED((N, D), jnp.float32),
        pltpu.SemaphoreType.DMA,
        pltpu.SemaphoreType.DMA,
    ],
)
def _sc_cnt(dst, z128, ones_h,
            cnt_out,
            didxa, didxb, ones_v, cstage, cnts, semia, semib):
    cid = lax.axis_index("c")
    sid = lax.axis_index("s")
    wid = cid * NS + sid
    rbase = sid * RPT
    pltpu.sync_copy(z128.at[pl.ds(0, SCH)], cstage)
    for j in range(NSCH):
        pltpu.sync_copy(cstage, cnts.at[pl.ds(rbase + j * SCH, SCH)])

    @pl.when(sid == NS - 1)
    def _tail_init():
        t0 = NS * RPT
        pltpu.sync_copy(cstage.at[pl.ds(0, RTAIL)], cnts.at[pl.ds(t0, RTAIL)])

    pltpu.sync_copy(ones_h, ones_v)
    plsc.subcore_barrier()
    e0 = wid * EPW
    dmyi = dst.at[pl.ds(0, K)]

    pltpu.async_copy(dst.at[pl.ds(e0, K)], didxa, semia)
    pltpu.async_copy(dst.at[pl.ds(e0 + K, K)], didxb, semib)

    def pair(j, carry):
        c = 2 * j
        pltpu.make_async_copy(dmyi, didxa, semia).wait()
        pltpu.sync_copy(ones_v, cnts.at[didxa], add=True)
        pltpu.async_copy(dst.at[pl.ds(e0 + (c + 2) * K, K)], didxa, semia)
        pltpu.make_async_copy(dmyi, didxb, semib).wait()
        pltpu.sync_copy(ones_v, cnts.at[didxb], add=True)
        pltpu.async_copy(dst.at[pl.ds(e0 + (c + 3) * K, K)], didxb, semib)
        return carry

    lax.fori_loop(0, (NCHUNK - 2) // 2, pair, 0)
    pltpu.make_async_copy(dmyi, didxa, semia).wait()
    pltpu.sync_copy(ones_v, cnts.at[didxa], add=True)
    pltpu.make_async_copy(dmyi, didxb, semib).wait()
    pltpu.sync_copy(ones_v, cnts.at[didxb], add=True)
    plsc.subcore_barrier()
    obase = cid * N
    for j in range(NSCH):
        o = rbase + j * SCH
        pltpu.sync_copy(cnts.at[pl.ds(o, SCH)], cstage)
        pltpu.sync_copy(cstage, cnt_out.at[pl.ds(obase + o, SCH)])

    @pl.when(sid == NS - 1)
    def _tail_out():
        t0 = NS * RPT
        pltpu.sync_copy(cnts.at[pl.ds(t0, RTAIL)], cstage.at[pl.ds(0, RTAIL)])
        pltpu.sync_copy(cstage.at[pl.ds(0, RTAIL)],
                        cnt_out.at[pl.ds(obase + t0, RTAIL)])


R = 400
G = N // R


def _dotT(a, w):
    return lax.dot_general(a, w, (((1,), (1,)), ((), ())),
                           preferred_element_type=jnp.float32)


def _dense1_body(xr, aggr, cntr, wl, wr, br, hr):
    agg = aggr[0] + aggr[1]
    c = cntr[0][:, 0:1] + cntr[1][:, 0:1]
    mean = agg / jnp.maximum(c, 1.0)
    o = _dotT(mean, wl[...]) + _dotT(xr[...], wr[...]) + br[...]
    n2 = jnp.sum(o * o, axis=1, keepdims=True)
    o = o * lax.rsqrt(jnp.maximum(n2, 1e-24))
    hr[...] = jnp.maximum(o, 0.0)


def _tc_dense1(x, aggp, cntp, Wl, Wr, b2d):
    return pl.pallas_call(
        _dense1_body,
        grid=(G,),
        in_specs=[
            pl.BlockSpec((R, D), lambda i: (i, 0)),
            pl.BlockSpec((NC, R, D), lambda i: (0, i, 0)),
            pl.BlockSpec((NC, R, D), lambda i: (0, i, 0)),
            pl.BlockSpec((H, D), lambda i: (0, 0)),
            pl.BlockSpec((H, D), lambda i: (0, 0)),
            pl.BlockSpec((1, H), lambda i: (0, 0)),
        ],
        out_specs=pl.BlockSpec((R, H), lambda i: (i, 0)),
        out_shape=jax.ShapeDtypeStruct((N, H), jnp.float32),
    )(x, aggp, cntp, Wl, Wr, b2d)


def _dense2_body(hr, aggr, cntr, batr, zr, wl, wr, br,
                 w1g, w1z, b1r, w2, b2r, wp, bpr, outr, pooled):
    i = pl.program_id(0)

    @pl.when(i == 0)
    def _init():
        pooled[...] = jnp.zeros_like(pooled)

    agg = aggr[0] + aggr[1]
    c = cntr[0][:, 0:1] + cntr[1][:, 0:1]
    mean = agg / jnp.maximum(c, 1.0)
    o = _dotT(mean, wl[...]) + _dotT(hr[...], wr[...]) + br[...]
    n2 = jnp.sum(o * o, axis=1, keepdims=True)
    o = o * lax.rsqrt(jnp.maximum(n2, 1e-24))
    h2 = jnp.maximum(o, 0.0)

    oh = (batr[...] == lax.broadcasted_iota(jnp.int32, (R, B), 1)
          ).astype(jnp.float32)
    pooled[...] += lax.dot_general(oh, h2, (((0,), (0,)), ((), ())),
                                   preferred_element_type=jnp.float32)

    @pl.when(i == G - 1)
    def _head():
        g = pooled[...]
        f = jnp.maximum(_dotT(g, w1g[...]) + _dotT(zr[...], w1z[...])
                        + b1r[...], 0.0)
        f = jnp.maximum(_dotT(f, w2[...]) + b2r[...], 0.0)
        out = jnp.sum(f * wp[...], axis=1, keepdims=True) + bpr[...]
        t = -out
        outr[...] = jnp.maximum(t, 0.0) + jnp.log(1.0 + jnp.exp(-jnp.abs(t)))


def _tc_dense2(h1, aggp, cntp, bat2d, z, Wl, Wr, b2d,
               W1g, W1z, bfc1_2d, Wfc2, bfc2_2d, Wp, bp2d):
    return pl.pallas_call(
        _dense2_body,
        grid=(G,),
        in_specs=[
            pl.BlockSpec((R, H), lambda i: (i, 0)),
            pl.BlockSpec((NC, R, H), lambda i: (0, i, 0)),
            pl.BlockSpec((NC, R, D), lambda i: (0, i, 0)),
            pl.BlockSpec((R, 1), lambda i: (i, 0)),
            pl.BlockSpec((B, SF), lambda i: (0, 0)),
            pl.BlockSpec((H, H), lambda i: (0, 0)),
            pl.BlockSpec((H, H), lambda i: (0, 0)),
            pl.BlockSpec((1, H), lambda i: (0, 0)),
            pl.BlockSpec((FC, H), lambda i: (0, 0)),
            pl.BlockSpec((FC, SF), lambda i: (0, 0)),
            pl.BlockSpec((1, FC), lambda i: (0, 0)),
            pl.BlockSpec((FC, FC), lambda i: (0, 0)),
            pl.BlockSpec((1, FC), lambda i: (0, 0)),
            pl.BlockSpec((1, FC), lambda i: (0, 0)),
            pl.BlockSpec((B, 1), lambda i: (0, 0)),
        ],
        out_specs=pl.BlockSpec((B, 1), lambda i: (0, 0)),
        out_shape=jax.ShapeDtypeStruct((B, 1), jnp.float32),
        scratch_shapes=[pltpu.VMEM((B, H), jnp.float32)],
    )(h1, aggp, cntp, bat2d, z, Wl, Wr, b2d,
      W1g, W1z, bfc1_2d, Wfc2, bfc2_2d, Wp, bp2d)


def kernel(x, edge_index, batch, z, Wl1, b1, Wr1, Wl2, b2, Wr2,
           Wfc1, bfc1, Wfc2, bfc2, Wp, bp):
    src = edge_index[0]
    dst = edge_index[1]
    x = x.astype(jnp.float32)
    z128 = jnp.zeros((N, D), jnp.float32)
    ones_h = jnp.ones((K, D), jnp.float32)

    cntp = _sc_cnt(dst, z128, ones_h).reshape(NC, N, D)
    aggp1 = _sc_agg(x, src, dst, z128).reshape(NC, N, D)
    h1 = _tc_dense1(x, aggp1, cntp, Wl1, Wr1, b1.reshape(1, H))
    aggp2 = _sc_agg(h1, src, dst, z128).reshape(NC, N, D)
    out = _tc_dense2(
        h1, aggp2, cntp, batch.reshape(N, 1), z, Wl2, Wr2, b2.reshape(1, H),
        Wfc1[:, :H], Wfc1[:, H:], bfc1.reshape(1, FC),
        Wfc2, bfc2.reshape(1, FC), Wp,
        jnp.broadcast_to(bp.reshape(1, 1), (B, 1)))
    return out


# speedup vs baseline: 8.8002x; 1.2619x over previous
import functools

import jax
import jax.numpy as jnp
from jax import lax
from jax.experimental import pallas as pl
from jax.experimental.pallas import tpu as pltpu
from jax.experimental.pallas import tpu_sc as plsc

N = 10000
E = 320000
D = 128
H = 128
FC = 128
B = 64
SF = 6

NC = 2
NS = 16
NW = NC * NS
EPW = E // NW
K = 80
NCHUNK = EPW // K
RPT = 624
RTAIL = N - NS * RPT
SCH = 24
NSCH = RPT // SCH

_mesh = plsc.VectorSubcoreMesh(core_axis_name="c", subcore_axis_name="s")


def _make_sc_agg(width):
    @functools.partial(
        pl.kernel,
        out_type=jax.ShapeDtypeStruct((NC * N, width), jnp.float32),
        mesh=_mesh,
        scratch_types=[
            pltpu.VMEM((EPW,), jnp.int32),
            pltpu.VMEM((K,), jnp.int32),
            pltpu.VMEM((K,), jnp.int32),
            pltpu.VMEM((K, width), jnp.float32),
            pltpu.VMEM((K, width), jnp.float32),
            pltpu.VMEM_SHARED((N, width), jnp.float32),
            pltpu.SemaphoreType.DMA,
            pltpu.SemaphoreType.DMA,
            pltpu.SemaphoreType.DMA,
            pltpu.SemaphoreType.DMA,
        ],
    )
    def _sc_agg(table, src, dst, zinit,
                agg_out,
                sidxall, didxa, didxb, rowsa, rowsb, aggs,
                semra, semrb, semia, semib):
        cid = lax.axis_index("c")
        sid = lax.axis_index("s")
        wid = cid * NS + sid
        rbase = sid * RPT
        stage = rowsa.at[pl.ds(0, SCH)]
        pltpu.sync_copy(zinit.at[pl.ds(0, SCH)], stage)
        for j in range(NSCH):
            pltpu.sync_copy(stage, aggs.at[pl.ds(rbase + j * SCH, SCH)])

        @pl.when(sid == NS - 1)
        def _tail_init():
            t0 = NS * RPT
            pltpu.sync_copy(rowsa.at[pl.ds(0, RTAIL)],
                            aggs.at[pl.ds(t0, RTAIL)])

        e0 = wid * EPW
        pltpu.sync_copy(src.at[pl.ds(e0, EPW)], sidxall)
        plsc.subcore_barrier()
        dmyr = table.at[pl.ds(0, K)]
        dmyi = dst.at[pl.ds(0, K)]

        pltpu.async_copy(dst.at[pl.ds(e0, K)], didxa, semia)
        pltpu.async_copy(dst.at[pl.ds(e0 + K, K)], didxb, semib)
        pltpu.async_copy(table.at[sidxall.at[pl.ds(0, K)]], rowsa, semra)

        def pair(j, carry):
            c = 2 * j
            pltpu.async_copy(table.at[sidxall.at[pl.ds((c + 1) * K, K)]],
                             rowsb, semrb)
            pltpu.make_async_copy(dmyr, rowsa, semra).wait()
            pltpu.make_async_copy(dmyi, didxa, semia).wait()
            pltpu.sync_copy(rowsa, aggs.at[didxa], add=True)
            pltpu.async_copy(table.at[sidxall.at[pl.ds((c + 2) * K, K)]],
                             rowsa, semra)
            pltpu.async_copy(dst.at[pl.ds(e0 + (c + 2) * K, K)], didxa, semia)
            pltpu.make_async_copy(dmyr, rowsb, semrb).wait()
            pltpu.make_async_copy(dmyi, didxb, semib).wait()
            pltpu.sync_copy(rowsb, aggs.at[didxb], add=True)
            pltpu.async_copy(dst.at[pl.ds(e0 + (c + 3) * K, K)], didxb, semib)
            return carry

        lax.fori_loop(0, (NCHUNK - 3) // 2, pair, 0)
        pltpu.async_copy(table.at[sidxall.at[pl.ds((NCHUNK - 2) * K, K)]],
                         rowsb, semrb)
        pltpu.make_async_copy(dmyr, rowsa, semra).wait()
        pltpu.make_async_copy(dmyi, didxa, semia).wait()
        pltpu.sync_copy(rowsa, aggs.at[didxa], add=True)
        pltpu.async_copy(table.at[sidxall.at[pl.ds((NCHUNK - 1) * K, K)]],
                         rowsa, semra)
        pltpu.async_copy(dst.at[pl.ds(e0 + (NCHUNK - 1) * K, K)], didxa,
                         semia)
        pltpu.make_async_copy(dmyr, rowsb, semrb).wait()
        pltpu.make_async_copy(dmyi, didxb, semib).wait()
        pltpu.sync_copy(rowsb, aggs.at[didxb], add=True)
        pltpu.make_async_copy(dmyr, rowsa, semra).wait()
        pltpu.make_async_copy(dmyi, didxa, semia).wait()
        pltpu.sync_copy(rowsa, aggs.at[didxa], add=True)
        plsc.subcore_barrier()
        obase = cid * N
        for j in range(NSCH):
            o = rbase + j * SCH
            pltpu.sync_copy(aggs.at[pl.ds(o, SCH)], stage)
            pltpu.sync_copy(stage, agg_out.at[pl.ds(obase + o, SCH)])

        @pl.when(sid == NS - 1)
        def _tail_out():
            t0 = NS * RPT
            pltpu.sync_copy(aggs.at[pl.ds(t0, RTAIL)],
                            rowsa.at[pl.ds(0, RTAIL)])
            pltpu.sync_copy(rowsa.at[pl.ds(0, RTAIL)],
                            agg_out.at[pl.ds(obase + t0, RTAIL)])

    return _sc_agg


_sc_agg = _make_sc_agg(D)


@functools.partial(
    pl.kernel,
    out_type=jax.ShapeDtypeStruct((NC * N, D), jnp.float32),
    mesh=_mesh,
    scratch_types=[
        pltpu.VMEM((K,), jnp.int32),
        pltpu.VMEM((K,), jnp.int32),
        pltpu.VMEM((K, D), jnp.float32),
        pltpu.VMEM((SCH, D), jnp.float32),
        pltpu.VMEM_SHARED((N, D), jnp.float32),
        pltpu.SemaphoreType.DMA,
        pltpu.SemaphoreType.DMA,
    ],
)
def _sc_cnt(dst, z128, ones_h,
            cnt_out,
            didxa, didxb, ones_v, cstage, cnts, semia, semib):
    cid = lax.axis_index("c")
    sid = lax.axis_index("s")
    wid = cid * NS + sid
    rbase = sid * RPT
    pltpu.sync_copy(z128.at[pl.ds(0, SCH)], cstage)
    for j in range(NSCH):
        pltpu.sync_copy(cstage, cnts.at[pl.ds(rbase + j * SCH, SCH)])

    @pl.when(sid == NS - 1)
    def _tail_init():
        t0 = NS * RPT
        pltpu.sync_copy(cstage.at[pl.ds(0, RTAIL)], cnts.at[pl.ds(t0, RTAIL)])

    pltpu.sync_copy(ones_h, ones_v)
    plsc.subcore_barrier()
    e0 = wid * EPW
    dmyi = dst.at[pl.ds(0, K)]

    pltpu.async_copy(dst.at[pl.ds(e0, K)], didxa, semia)
    pltpu.async_copy(dst.at[pl.ds(e0 + K, K)], didxb, semib)

    def pair(j, carry):
        c = 2 * j
        pltpu.make_async_copy(dmyi, didxa, semia).wait()
        pltpu.sync_copy(ones_v, cnts.at[didxa], add=True)
        pltpu.async_copy(dst.at[pl.ds(e0 + (c + 2) * K, K)], didxa, semia)
        pltpu.make_async_copy(dmyi, didxb, semib).wait()
        pltpu.sync_copy(ones_v, cnts.at[didxb], add=True)
        pltpu.async_copy(dst.at[pl.ds(e0 + (c + 3) * K, K)], didxb, semib)
        return carry

    lax.fori_loop(0, (NCHUNK - 3) // 2, pair, 0)
    pltpu.make_async_copy(dmyi, didxa, semia).wait()
    pltpu.sync_copy(ones_v, cnts.at[didxa], add=True)
    pltpu.async_copy(dst.at[pl.ds(e0 + (NCHUNK - 1) * K, K)], didxa, semia)
    pltpu.make_async_copy(dmyi, didxb, semib).wait()
    pltpu.sync_copy(ones_v, cnts.at[didxb], add=True)
    pltpu.make_async_copy(dmyi, didxa, semia).wait()
    pltpu.sync_copy(ones_v, cnts.at[didxa], add=True)
    plsc.subcore_barrier()
    obase = cid * N
    for j in range(NSCH):
        o = rbase + j * SCH
        pltpu.sync_copy(cnts.at[pl.ds(o, SCH)], cstage)
        pltpu.sync_copy(cstage, cnt_out.at[pl.ds(obase + o, SCH)])

    @pl.when(sid == NS - 1)
    def _tail_out():
        t0 = NS * RPT
        pltpu.sync_copy(cnts.at[pl.ds(t0, RTAIL)], cstage.at[pl.ds(0, RTAIL)])
        pltpu.sync_copy(cstage.at[pl.ds(0, RTAIL)],
                        cnt_out.at[pl.ds(obase + t0, RTAIL)])


R = 400
G = N // R


def _dotT(a, w):
    return lax.dot_general(a, w, (((1,), (1,)), ((), ())),
                           preferred_element_type=jnp.float32)


def _dense1_body(xr, aggr, cntr, wl, wr, br, hr):
    agg = aggr[0] + aggr[1]
    c = cntr[0][:, 0:1] + cntr[1][:, 0:1]
    mean = agg / jnp.maximum(c, 1.0)
    o = _dotT(mean, wl[...]) + _dotT(xr[...], wr[...]) + br[...]
    n2 = jnp.sum(o * o, axis=1, keepdims=True)
    o = o * lax.rsqrt(jnp.maximum(n2, 1e-24))
    hr[...] = jnp.maximum(o, 0.0)


def _tc_dense1(x, aggp, cntp, Wl, Wr, b2d):
    return pl.pallas_call(
        _dense1_body,
        grid=(G,),
        in_specs=[
            pl.BlockSpec((R, D), lambda i: (i, 0)),
            pl.BlockSpec((NC, R, D), lambda i: (0, i, 0)),
            pl.BlockSpec((NC, R, D), lambda i: (0, i, 0)),
            pl.BlockSpec((H, D), lambda i: (0, 0)),
            pl.BlockSpec((H, D), lambda i: (0, 0)),
            pl.BlockSpec((1, H), lambda i: (0, 0)),
        ],
        out_specs=pl.BlockSpec((R, H), lambda i: (i, 0)),
        out_shape=jax.ShapeDtypeStruct((N, H), jnp.float32),
    )(x, aggp, cntp, Wl, Wr, b2d)


def _dense2_body(hr, aggr, cntr, batr, zr, wl, wr, br,
                 w1g, w1z, b1r, w2, b2r, wp, bpr, outr, pooled):
    i = pl.program_id(0)

    @pl.when(i == 0)
    def _init():
        pooled[...] = jnp.zeros_like(pooled)

    agg = aggr[0] + aggr[1]
    c = cntr[0][:, 0:1] + cntr[1][:, 0:1]
    mean = agg / jnp.maximum(c, 1.0)
    o = _dotT(mean, wl[...]) + _dotT(hr[...], wr[...]) + br[...]
    n2 = jnp.sum(o * o, axis=1, keepdims=True)
    o = o * lax.rsqrt(jnp.maximum(n2, 1e-24))
    h2 = jnp.maximum(o, 0.0)

    oh = (batr[...] == lax.broadcasted_iota(jnp.int32, (R, B), 1)
          ).astype(jnp.float32)
    pooled[...] += lax.dot_general(oh, h2, (((0,), (0,)), ((), ())),
                                   preferred_element_type=jnp.float32)

    @pl.when(i == G - 1)
    def _head():
        g = pooled[...]
        f = jnp.maximum(_dotT(g, w1g[...]) + _dotT(zr[...], w1z[...])
                        + b1r[...], 0.0)
        f = jnp.maximum(_dotT(f, w2[...]) + b2r[...], 0.0)
        out = jnp.sum(f * wp[...], axis=1, keepdims=True) + bpr[...]
        t = -out
        outr[...] = jnp.maximum(t, 0.0) + jnp.log(1.0 + jnp.exp(-jnp.abs(t)))


def _tc_dense2(h1, aggp, cntp, bat2d, z, Wl, Wr, b2d,
               W1g, W1z, bfc1_2d, Wfc2, bfc2_2d, Wp, bp2d):
    return pl.pallas_call(
        _dense2_body,
        grid=(G,),
        in_specs=[
            pl.BlockSpec((R, H), lambda i: (i, 0)),
            pl.BlockSpec((NC, R, H), lambda i: (0, i, 0)),
            pl.BlockSpec((NC, R, D), lambda i: (0, i, 0)),
            pl.BlockSpec((R, 1), lambda i: (i, 0)),
            pl.BlockSpec((B, SF), lambda i: (0, 0)),
            pl.BlockSpec((H, H), lambda i: (0, 0)),
            pl.BlockSpec((H, H), lambda i: (0, 0)),
            pl.BlockSpec((1, H), lambda i: (0, 0)),
            pl.BlockSpec((FC, H), lambda i: (0, 0)),
            pl.BlockSpec((FC, SF), lambda i: (0, 0)),
            pl.BlockSpec((1, FC), lambda i: (0, 0)),
            pl.BlockSpec((FC, FC), lambda i: (0, 0)),
            pl.BlockSpec((1, FC), lambda i: (0, 0)),
            pl.BlockSpec((1, FC), lambda i: (0, 0)),
            pl.BlockSpec((B, 1), lambda i: (0, 0)),
        ],
        out_specs=pl.BlockSpec((B, 1), lambda i: (0, 0)),
        out_shape=jax.ShapeDtypeStruct((B, 1), jnp.float32),
        scratch_shapes=[pltpu.VMEM((B, H), jnp.float32)],
    )(h1, aggp, cntp, bat2d, z, Wl, Wr, b2d,
      W1g, W1z, bfc1_2d, Wfc2, bfc2_2d, Wp, bp2d)


def kernel(x, edge_index, batch, z, Wl1, b1, Wr1, Wl2, b2, Wr2,
           Wfc1, bfc1, Wfc2, bfc2, Wp, bp):
    src = edge_index[0]
    dst = edge_index[1]
    x = x.astype(jnp.float32)
    z128 = jnp.zeros((N, D), jnp.float32)
    ones_h = jnp.ones((K, D), jnp.float32)

    cntp = _sc_cnt(dst, z128, ones_h).reshape(NC, N, D)
    aggp1 = _sc_agg(x, src, dst, z128).reshape(NC, N, D)
    h1 = _tc_dense1(x, aggp1, cntp, Wl1, Wr1, b1.reshape(1, H))
    aggp2 = _sc_agg(h1, src, dst, z128).reshape(NC, N, D)
    out = _tc_dense2(
        h1, aggp2, cntp, batch.reshape(N, 1), z, Wl2, Wr2, b2.reshape(1, H),
        Wfc1[:, :H], Wfc1[:, H:], bfc1.reshape(1, FC),
        Wfc2, bfc2.reshape(1, FC), Wp,
        jnp.broadcast_to(bp.reshape(1, 1), (B, 1)))
    return out
